# baseline (speedup 1.0000x reference)
import functools

import jax
import jax.numpy as jnp
from jax import lax
from jax.experimental import pallas as pl
from jax.experimental.pallas import tpu as pltpu
from jax.experimental.pallas import tpu_sc as plsc

HIDDEN = 2048
NUM_CB = 32
L = 16
NWORK = 32
TOK = 4096
TPW = TOK // NWORK
GRP = 8
NGRP = TPW // GRP
NHID = HIDDEN // L
UR = 16
UPT = NUM_CB // UR


def _sc_embed(ids_audio, ids_text, text_table, audio_table, offsets):
    mesh = plsc.VectorSubcoreMesh(core_axis_name="c", subcore_axis_name="s")

    @functools.partial(
        pl.kernel,
        mesh=mesh,
        out_type=jax.ShapeDtypeStruct((TOK, HIDDEN), jnp.float32),
        scratch_types=[
            pltpu.VMEM((TPW * NUM_CB,), jnp.int32),
            pltpu.VMEM((TPW,), jnp.int32),
            pltpu.VMEM((NUM_CB,), jnp.int32),
            pltpu.VMEM((GRP, HIDDEN), jnp.float32),
            pltpu.VMEM((UR, HIDDEN), jnp.float32),
            pltpu.VMEM((UR, HIDDEN), jnp.float32),
            pltpu.VMEM((2, 1, HIDDEN), jnp.float32),
            pltpu.SemaphoreType.DMA,
            pltpu.SemaphoreType.DMA,
            pltpu.SemaphoreType.DMA,
            pltpu.SemaphoreType.DMA,
            pltpu.SemaphoreType.DMA,
        ],
    )
    def body(ids_audio_h, ids_text_h, ttab_h, atab_h, offs_h, out_h,
             aidx_v, tid_v, offs_v, tb, b0, b1, acc,
             sem_a0, sem_a1, sem_t, sem_o0, sem_o1):
        wid = lax.axis_index("s") * 2 + lax.axis_index("c")
        base = wid * TPW

        pltpu.sync_copy(ids_audio_h.at[pl.ds(base * NUM_CB, TPW * NUM_CB)], aidx_v)
        pltpu.sync_copy(ids_text_h.at[pl.ds(base, TPW)], tid_v)
        pltpu.sync_copy(offs_h, offs_v)

        pltpu.async_copy(ttab_h.at[tid_v.at[pl.ds(0, GRP)]], tb, sem_t)

        zeros = jnp.zeros((L,), jnp.int32)
        offs01 = offs_v[pl.ds(0, L)]
        offs23 = offs_v[pl.ds(L, L)]

        def cidx(t, carry):
            tok01 = aidx_v[pl.ds(NUM_CB * t, L)]
            tok23 = aidx_v[pl.ds(NUM_CB * t + L, L)]
            aidx_v[pl.ds(NUM_CB * t, L)] = jnp.where(tok01 == 0, zeros, tok01 + offs01)
            aidx_v[pl.ds(NUM_CB * t + L, L)] = jnp.where(tok23 == 0, zeros, tok23 + offs23)
            return carry
        lax.fori_loop(0, TPW, cidx, 0)

        pltpu.async_copy(atab_h.at[aidx_v.at[pl.ds(0, UR)]], b0, sem_a0)

        def tok_body(t, carry):
            g = t // GRP
            gl = t % GRP
            po = t % 2

            @pl.when(gl == 0)
            def _():
                pltpu.make_async_copy(ttab_h.at[pl.ds(0, GRP)], tb, sem_t).wait()

            @pl.when(jnp.logical_and(po == 0, t >= 2))
            def _():
                pltpu.make_async_copy(out_h.at[pl.ds(0, 1)], acc.at[0], sem_o0).wait()

            @pl.when(jnp.logical_and(po == 1, t >= 2))
            def _():
                pltpu.make_async_copy(out_h.at[pl.ds(0, 1)], acc.at[1], sem_o1).wait()

            u = UPT * t
            pltpu.async_copy(atab_h.at[aidx_v.at[pl.ds((u + 1) * UR, UR)]], b1, sem_a1)
            pltpu.make_async_copy(atab_h.at[pl.ds(0, UR)], b0, sem_a0).wait()

            def acc0(c, carry2):
                for k in range(4):
                    cs = pl.ds((4 * c + k) * L, L)
                    vals = [tb[gl, cs]] + [b0[r, cs] for r in range(UR)]
                    while len(vals) > 1:
                        vals = [a + b for a, b in zip(vals[::2], vals[1::2])] \
                            + ([vals[-1]] if len(vals) % 2 else [])
                    acc[po, 0, cs] = vals[0]
                return carry2
            lax.fori_loop(0, NHID // 4, acc0, 0)

            @pl.when(jnp.logical_and(gl == GRP - 1, g + 1 < NGRP))
            def _():
                pltpu.async_copy(
                    ttab_h.at[tid_v.at[pl.ds((g + 1) * GRP, GRP)]], tb, sem_t)

            @pl.when(t + 1 < TPW)
            def _():
                pltpu.async_copy(
                    atab_h.at[aidx_v.at[pl.ds((u + 2) * UR, UR)]], b0, sem_a0)

            pltpu.make_async_copy(atab_h.at[pl.ds(0, UR)], b1, sem_a1).wait()

            def acc1(c, carry2):
                for k in range(4):
                    cs = pl.ds((4 * c + k) * L, L)
                    vals = [acc[po, 0, cs]] + [b1[r, cs] for r in range(UR)]
                    while len(vals) > 1:
                        vals = [a + b for a, b in zip(vals[::2], vals[1::2])] \
                            + ([vals[-1]] if len(vals) % 2 else [])
                    acc[po, 0, cs] = vals[0]
                return carry2
            lax.fori_loop(0, NHID // 4, acc1, 0)

            @pl.when(po == 0)
            def _():
                pltpu.async_copy(acc.at[0], out_h.at[pl.ds(base + t, 1)], sem_o0)

            @pl.when(po == 1)
            def _():
                pltpu.async_copy(acc.at[1], out_h.at[pl.ds(base + t, 1)], sem_o1)

            return carry
        lax.fori_loop(0, TPW, tok_body, 0)

        pltpu.make_async_copy(out_h.at[pl.ds(0, 1)], acc.at[0], sem_o0).wait()
        pltpu.make_async_copy(out_h.at[pl.ds(0, 1)], acc.at[1], sem_o1).wait()

    return body(ids_audio, ids_text, text_table, audio_table, offsets)


def kernel(input_ids, text_table, audio_table, audio_tokens_offsets):
    b, s, _ = input_ids.shape
    ids = input_ids.reshape(b * s, NUM_CB + 1).astype(jnp.int32)
    ids_audio = ids[:, :NUM_CB].reshape(TOK * NUM_CB)
    ids_text = ids[:, NUM_CB]
    offs = audio_tokens_offsets.astype(jnp.int32)
    out = _sc_embed(ids_audio, ids_text, text_table, audio_table, offs)
    return out.reshape(b, s, HIDDEN)
